# baseline (speedup 1.0000x reference)
import jax
import jax.numpy as jnp
from jax.experimental import pallas as pl
from jax.experimental.pallas import tpu as pltpu


def _support_body(x_ref, w_ref, out_ref):
    out_ref[...] = jnp.dot(x_ref[...], w_ref[...],
                           preferred_element_type=jnp.float32)


def _diffuse_body(t_ref, adj_ref, sup_ref, sup_rows_ref, out_ref):
    t = t_ref[0, 0]
    acc = jnp.dot(adj_ref[...], sup_ref[...],
                  preferred_element_type=jnp.float32)
    out_ref[...] = t * acc + (1.0 - t) * sup_rows_ref[...]


def kernel(x, adj, weight, t):
    n, in_f = x.shape
    out_f = weight.shape[1]

    support = pl.pallas_call(
        _support_body,
        out_shape=jax.ShapeDtypeStruct((n, out_f), jnp.float32),
    )(x, weight)

    bm = 400
    assert n % bm == 0
    grid = (n // bm,)
    t2 = t.reshape(1, 1).astype(jnp.float32)

    out = pl.pallas_call(
        _diffuse_body,
        grid=grid,
        in_specs=[
            pl.BlockSpec((1, 1), lambda i: (0, 0)),
            pl.BlockSpec((bm, n), lambda i: (i, 0)),
            pl.BlockSpec((n, out_f), lambda i: (0, 0)),
            pl.BlockSpec((bm, out_f), lambda i: (i, 0)),
        ],
        out_specs=pl.BlockSpec((bm, out_f), lambda i: (i, 0)),
        out_shape=jax.ShapeDtypeStruct((n, out_f), jnp.float32),
        compiler_params=pltpu.CompilerParams(
            dimension_semantics=("arbitrary",),
        ),
    )(t2, adj, support, support)
    return out


# speedup vs baseline: 1.0827x; 1.0827x over previous
import jax
import jax.numpy as jnp
from jax.experimental import pallas as pl
from jax.experimental.pallas import tpu as pltpu

_BM = 200


def _fused_body(t_ref, x_ref, w_ref, adj_ref, out_ref, sup_ref):
    i = pl.program_id(0)

    @pl.when(i == 0)
    def _():
        sup_ref[...] = jnp.dot(x_ref[...], w_ref[...],
                               preferred_element_type=jnp.float32)

    t = t_ref[0, 0]
    acc = jnp.dot(adj_ref[...], sup_ref[...],
                  preferred_element_type=jnp.float32)
    rows = sup_ref[pl.ds(i * _BM, _BM), :]
    out_ref[...] = t * acc + (1.0 - t) * rows


def kernel(x, adj, weight, t):
    n, in_f = x.shape
    out_f = weight.shape[1]
    assert n % _BM == 0
    grid = (n // _BM,)
    t2 = t.reshape(1, 1).astype(jnp.float32)

    out = pl.pallas_call(
        _fused_body,
        grid=grid,
        in_specs=[
            pl.BlockSpec((1, 1), lambda i: (0, 0)),
            pl.BlockSpec((n, in_f), lambda i: (0, 0)),
            pl.BlockSpec((in_f, out_f), lambda i: (0, 0)),
            pl.BlockSpec((_BM, n), lambda i: (i, 0)),
        ],
        out_specs=pl.BlockSpec((_BM, out_f), lambda i: (i, 0)),
        out_shape=jax.ShapeDtypeStruct((n, out_f), jnp.float32),
        scratch_shapes=[pltpu.VMEM((n, out_f), jnp.float32)],
        compiler_params=pltpu.CompilerParams(
            dimension_semantics=("arbitrary",),
        ),
    )(t2, x, weight, adj)
    return out
